# Initial kernel scaffold; baseline (speedup 1.0000x reference)
import jax
import jax.numpy as jnp
from jax.experimental import pallas as pl


def kernel(edge_features, edge_index, pos):
    raise NotImplementedError("write your pallas kernel here")


# speedup vs baseline: 4.4349x; 4.4349x over previous
import functools

import jax
import jax.numpy as jnp
from jax import lax
from jax.experimental import pallas as pl
from jax.experimental.pallas import tpu as pltpu
from jax.experimental.pallas import tpu_sc as plsc

N_NODES = 10000
N_EDGES = 320000
D_FEAT = 128
SCALE = float(32.0 ** -0.5)

NC = 2
NS = 16
NW = NC * NS
E_PER_W = N_EDGES // NW
CHUNK = 80
NCHUNK = E_PER_W // CHUNK
N_PAD = 10240
STRIPE = N_PAD // NS


def _sc_scatter_partials(edge_features, idx3, zeros):
    mesh = plsc.VectorSubcoreMesh(core_axis_name="c", subcore_axis_name="s")

    @functools.partial(
        pl.kernel,
        out_type=jax.ShapeDtypeStruct((NC, N_PAD, D_FEAT), jnp.float32),
        mesh=mesh,
        scratch_types=[
            pltpu.VMEM((NCHUNK, CHUNK), jnp.int32),
            pltpu.VMEM((CHUNK, D_FEAT), jnp.float32),
            pltpu.VMEM_SHARED((N_PAD, D_FEAT), jnp.float32),
        ],
    )
    def k(ef_hbm, idx_hbm, zeros_hbm, out_hbm, idx_v, buf_v, acc):
        c = lax.axis_index("c")
        s = lax.axis_index("s")
        wid = s * NC + c
        r0 = s * STRIPE
        pltpu.sync_copy(zeros_hbm, acc.at[pl.ds(r0, STRIPE)])
        pltpu.sync_copy(idx_hbm.at[wid], idx_v)
        plsc.subcore_barrier()

        ebase = wid * E_PER_W

        def body(j, carry):
            pltpu.sync_copy(ef_hbm.at[pl.ds(ebase + j * CHUNK, CHUNK)], buf_v)
            pltpu.sync_copy(buf_v, acc.at[idx_v.at[j]], add=True)
            return carry

        lax.fori_loop(0, NCHUNK, body, 0)
        plsc.subcore_barrier()
        pltpu.sync_copy(acc.at[pl.ds(r0, STRIPE)],
                        out_hbm.at[c, pl.ds(r0, STRIPE)])

    return k(edge_features, idx3, zeros)


def _tc_combine(partials):
    def body(p_ref, o_ref):
        o_ref[...] = (p_ref[0] + p_ref[1]) * SCALE

    return pl.pallas_call(
        body,
        out_shape=jax.ShapeDtypeStruct((N_NODES, D_FEAT), jnp.float32),
        grid=(10,),
        in_specs=[pl.BlockSpec((2, 1000, D_FEAT), lambda i: (0, i, 0))],
        out_specs=pl.BlockSpec((1000, D_FEAT), lambda i: (i, 0)),
    )(partials)


def kernel(edge_features, edge_index, pos):
    idx3 = edge_index[0].reshape(NW, NCHUNK, CHUNK)
    zeros = jnp.zeros((STRIPE, D_FEAT), jnp.float32)
    partials = _sc_scatter_partials(edge_features, idx3, zeros)
    return _tc_combine(partials)


# speedup vs baseline: 6.9083x; 1.5577x over previous
import functools

import jax
import jax.numpy as jnp
from jax import lax
from jax.experimental import pallas as pl
from jax.experimental.pallas import tpu as pltpu
from jax.experimental.pallas import tpu_sc as plsc

N_NODES = 10000
N_EDGES = 320000
D_FEAT = 128
SCALE = float(32.0 ** -0.5)

NC = 2
NS = 16
NW = NC * NS
E_PER_W = N_EDGES // NW
CHUNK = 80
NCHUNK = E_PER_W // CHUNK
RB = CHUNK
NB = E_PER_W // RB
N_PAD = 10240
STRIPE = N_PAD // NS


def _sc_scatter_partials(edge_features, idx3, zeros):
    mesh = plsc.VectorSubcoreMesh(core_axis_name="c", subcore_axis_name="s")

    @functools.partial(
        pl.kernel,
        out_type=jax.ShapeDtypeStruct((NC, N_PAD, D_FEAT), jnp.float32),
        mesh=mesh,
        scratch_types=[
            pltpu.VMEM((NCHUNK, CHUNK), jnp.int32),
            pltpu.VMEM((RB, D_FEAT), jnp.float32),
            pltpu.VMEM((RB, D_FEAT), jnp.float32),
            pltpu.VMEM_SHARED((N_PAD, D_FEAT), jnp.float32),
            pltpu.SemaphoreType.DMA,
            pltpu.SemaphoreType.DMA,
        ],
    )
    def k(ef_hbm, idx_hbm, zeros_hbm, out_hbm, idx_v, buf0, buf1, acc,
          rsem0, rsem1):
        c = lax.axis_index("c")
        s = lax.axis_index("s")
        wid = s * NC + c
        r0 = s * STRIPE
        ebase = wid * E_PER_W
        bufs = (buf0, buf1)
        rsems = (rsem0, rsem1)

        def rd_slice(b):
            return ef_hbm.at[pl.ds(ebase + b * RB, RB)]

        pltpu.async_copy(rd_slice(0), buf0, rsem0)
        pltpu.async_copy(rd_slice(1), buf1, rsem1)
        pltpu.sync_copy(zeros_hbm, acc.at[pl.ds(r0, STRIPE)])
        pltpu.sync_copy(idx_hbm.at[wid], idx_v)
        plsc.subcore_barrier()

        def body(p, carry):
            for h in range(2):
                b = 2 * p + h
                buf, rsem = bufs[h], rsems[h]
                pltpu.make_async_copy(rd_slice(b), buf, rsem).wait()
                pltpu.sync_copy(buf, acc.at[idx_v.at[b]], add=True)
                @pl.when(b + 2 < NB)
                def _():
                    pltpu.async_copy(rd_slice(b + 2), buf, rsem)
            return carry

        lax.fori_loop(0, NB // 2, body, 0)
        pltpu.make_async_copy(rd_slice(NB - 1), buf0, rsem0).wait()
        pltpu.sync_copy(buf0, acc.at[idx_v.at[NB - 1]], add=True)
        plsc.subcore_barrier()
        pltpu.sync_copy(acc.at[pl.ds(r0, STRIPE)],
                        out_hbm.at[c, pl.ds(r0, STRIPE)])

    return k(edge_features, idx3, zeros)


def _tc_combine(partials):
    def body(p_ref, o_ref):
        o_ref[...] = (p_ref[0] + p_ref[1]) * SCALE

    return pl.pallas_call(
        body,
        out_shape=jax.ShapeDtypeStruct((N_NODES, D_FEAT), jnp.float32),
        grid=(10,),
        in_specs=[pl.BlockSpec((2, 1000, D_FEAT), lambda i: (0, i, 0))],
        out_specs=pl.BlockSpec((1000, D_FEAT), lambda i: (i, 0)),
    )(partials)


def kernel(edge_features, edge_index, pos):
    idx3 = edge_index[0].reshape(NW, NCHUNK, CHUNK)
    zeros = jnp.zeros((STRIPE, D_FEAT), jnp.float32)
    partials = _sc_scatter_partials(edge_features, idx3, zeros)
    return _tc_combine(partials)


# speedup vs baseline: 7.4655x; 1.0807x over previous
import functools

import jax
import jax.numpy as jnp
from jax import lax
from jax.experimental import pallas as pl
from jax.experimental.pallas import tpu as pltpu
from jax.experimental.pallas import tpu_sc as plsc

N_NODES = 10000
N_EDGES = 320000
D_FEAT = 128
SCALE = float(32.0 ** -0.5)

NC = 2
NS = 16
NW = NC * NS
E_PER_W = N_EDGES // NW
CHUNK = 80
NCHUNK = E_PER_W // CHUNK
RB = CHUNK
NB = E_PER_W // RB
N_PAD = 10240
STRIPE = N_PAD // NS


def _sc_scatter_partials(edge_features, idx3, zeros):
    mesh = plsc.VectorSubcoreMesh(core_axis_name="c", subcore_axis_name="s")

    @functools.partial(
        pl.kernel,
        out_type=jax.ShapeDtypeStruct((NC, N_PAD, D_FEAT), jnp.float32),
        mesh=mesh,
        scratch_types=[
            pltpu.VMEM((NCHUNK, CHUNK), jnp.int32),
            pltpu.VMEM((RB, D_FEAT), jnp.float32),
            pltpu.VMEM((RB, D_FEAT), jnp.float32),
            pltpu.VMEM_SHARED((N_PAD, D_FEAT), jnp.float32),
            pltpu.SemaphoreType.DMA,
            pltpu.SemaphoreType.DMA,
        ],
    )
    def k(ef_hbm, idx_hbm, zeros_hbm, out_hbm, idx_v, buf0, buf1, acc,
          rsem0, rsem1):
        c = lax.axis_index("c")
        s = lax.axis_index("s")
        wid = s * NC + c
        r0 = s * STRIPE
        ebase = wid * E_PER_W
        bufs = (buf0, buf1)
        rsems = (rsem0, rsem1)

        def rd_slice(b):
            return ef_hbm.at[pl.ds(ebase + b * RB, RB)]

        pltpu.async_copy(rd_slice(0), buf0, rsem0)
        pltpu.async_copy(rd_slice(1), buf1, rsem1)
        pltpu.sync_copy(zeros_hbm, acc.at[pl.ds(r0, STRIPE)])
        pltpu.sync_copy(idx_hbm.at[0, wid], idx_v)
        plsc.subcore_barrier()

        def body(p, carry):
            for h in range(2):
                b = 2 * p + h
                buf, rsem = bufs[h], rsems[h]
                pltpu.make_async_copy(rd_slice(b), buf, rsem).wait()
                pltpu.sync_copy(buf, acc.at[idx_v.at[b]], add=True)
                @pl.when(b + 2 < NB)
                def _():
                    pltpu.async_copy(rd_slice(b + 2), buf, rsem)
            return carry

        lax.fori_loop(0, NB // 2, body, 0)
        pltpu.make_async_copy(rd_slice(NB - 1), buf0, rsem0).wait()
        pltpu.sync_copy(buf0, acc.at[idx_v.at[NB - 1]], add=True)
        plsc.subcore_barrier()
        pltpu.sync_copy(acc.at[pl.ds(r0, STRIPE)],
                        out_hbm.at[c, pl.ds(r0, STRIPE)])

    return k(edge_features, idx3, zeros)


def _tc_combine(partials):
    def body(p_ref, o_ref):
        o_ref[...] = (p_ref[0] + p_ref[1]) * SCALE

    return pl.pallas_call(
        body,
        out_shape=jax.ShapeDtypeStruct((N_NODES, D_FEAT), jnp.float32),
        grid=(5,),
        in_specs=[pl.BlockSpec((2, 2000, D_FEAT), lambda i: (0, i, 0))],
        out_specs=pl.BlockSpec((2000, D_FEAT), lambda i: (i, 0)),
    )(partials)


def kernel(edge_features, edge_index, pos):
    idx4 = edge_index.reshape(2, NW, NCHUNK, CHUNK)
    zeros = jnp.zeros((STRIPE, D_FEAT), jnp.float32)
    partials = _sc_scatter_partials(edge_features, idx4, zeros)
    return _tc_combine(partials)


# speedup vs baseline: 8.4633x; 1.1336x over previous
import functools

import jax
import jax.numpy as jnp
from jax import lax
from jax.experimental import pallas as pl
from jax.experimental.pallas import tpu as pltpu
from jax.experimental.pallas import tpu_sc as plsc

N_NODES = 10000
N_EDGES = 320000
D_FEAT = 128
SCALE = float(32.0 ** -0.5)

NC = 2
NS = 16
NW = NC * NS
CH = 128
NCHUNKS = N_EDGES // CH
CPT = NCHUNKS // NW
NEXTRA = NCHUNKS - CPT * NW
N_PAD = 10240
STRIPE = N_PAD // NS


def _sc_scatter_partials(edge_features, edge_index, zeros):
    mesh = plsc.VectorSubcoreMesh(core_axis_name="c", subcore_axis_name="s")

    @functools.partial(
        pl.kernel,
        out_type=jax.ShapeDtypeStruct((NC, N_PAD, D_FEAT), jnp.float32),
        mesh=mesh,
        scratch_types=[
            pltpu.VMEM((CH,), jnp.int32),
            pltpu.VMEM((CH,), jnp.int32),
            pltpu.VMEM((CH, D_FEAT), jnp.float32),
            pltpu.VMEM((CH, D_FEAT), jnp.float32),
            pltpu.VMEM_SHARED((N_PAD, D_FEAT), jnp.float32),
            pltpu.SemaphoreType.DMA,
            pltpu.SemaphoreType.DMA,
            pltpu.SemaphoreType.DMA,
            pltpu.SemaphoreType.DMA,
        ],
    )
    def k(ef_hbm, ei_hbm, zeros_hbm, out_hbm, ir0, ir1, b0, b1, acc,
          isem0, isem1, rsem0, rsem1):
        c = lax.axis_index("c")
        s = lax.axis_index("s")
        wid = s * NC + c
        r0 = s * STRIPE
        cbase = wid * CPT
        irs, bufs = (ir0, ir1), (b0, b1)
        isems, rsems = (isem0, isem1), (rsem0, rsem1)

        def idx_slice(cid):
            return ei_hbm.at[0, pl.ds(cid * CH, CH)]

        def edge_slice(cid):
            return ef_hbm.at[pl.ds(cid * CH, CH)]

        for h in range(2):
            pltpu.async_copy(idx_slice(cbase + h), irs[h], isems[h])
            pltpu.async_copy(edge_slice(cbase + h), bufs[h], rsems[h])
        pltpu.sync_copy(zeros_hbm, acc.at[pl.ds(r0, STRIPE)])
        plsc.subcore_barrier()

        def body(p, carry):
            for h in range(2):
                j = 2 * p + h
                cid = cbase + j
                ir, buf = irs[h], bufs[h]
                pltpu.make_async_copy(idx_slice(cid), ir, isems[h]).wait()
                pltpu.make_async_copy(edge_slice(cid), buf, rsems[h]).wait()
                pltpu.sync_copy(buf, acc.at[ir], add=True)
                @pl.when(j + 2 < CPT)
                def _():
                    pltpu.async_copy(idx_slice(cid + 2), ir, isems[h])
                    pltpu.async_copy(edge_slice(cid + 2), buf, rsems[h])
            return carry

        lax.fori_loop(0, CPT // 2, body, 0)

        @pl.when(wid < NEXTRA)
        def _():
            cid = NW * CPT + wid
            pltpu.sync_copy(idx_slice(cid), ir0)
            pltpu.sync_copy(edge_slice(cid), b0)
            pltpu.sync_copy(b0, acc.at[ir0], add=True)

        plsc.subcore_barrier()
        pltpu.sync_copy(acc.at[pl.ds(r0, STRIPE)],
                        out_hbm.at[c, pl.ds(r0, STRIPE)])

    return k(edge_features, edge_index, zeros)


def _tc_combine(partials):
    def body(p_ref, o_ref):
        o_ref[...] = (p_ref[0] + p_ref[1]) * SCALE

    return pl.pallas_call(
        body,
        out_shape=jax.ShapeDtypeStruct((N_NODES, D_FEAT), jnp.float32),
        grid=(5,),
        in_specs=[pl.BlockSpec((2, 2000, D_FEAT), lambda i: (0, i, 0))],
        out_specs=pl.BlockSpec((2000, D_FEAT), lambda i: (i, 0)),
    )(partials)


def kernel(edge_features, edge_index, pos):
    zeros = jnp.zeros((STRIPE, D_FEAT), jnp.float32)
    partials = _sc_scatter_partials(edge_features, edge_index, zeros)
    return _tc_combine(partials)


# speedup vs baseline: 8.6274x; 1.0194x over previous
import functools

import jax
import jax.numpy as jnp
from jax import lax
from jax.experimental import pallas as pl
from jax.experimental.pallas import tpu as pltpu
from jax.experimental.pallas import tpu_sc as plsc

N_NODES = 10000
N_EDGES = 320000
D_FEAT = 128
SCALE = float(32.0 ** -0.5)

NC = 2
NS = 16
NW = NC * NS
CH = 128
NCHUNKS = N_EDGES // CH
CPT = NCHUNKS // NW
NEXTRA = NCHUNKS - CPT * NW
N_PAD = 10112
STRIPE = N_PAD // NS


def _sc_scatter_partials(edge_features, edge_index, zeros):
    mesh = plsc.VectorSubcoreMesh(core_axis_name="c", subcore_axis_name="s")

    @functools.partial(
        pl.kernel,
        out_type=jax.ShapeDtypeStruct((NC, N_PAD, D_FEAT), jnp.float32),
        mesh=mesh,
        scratch_types=[
            pltpu.VMEM((CH,), jnp.int32),
            pltpu.VMEM((CH,), jnp.int32),
            pltpu.VMEM((CH, D_FEAT), jnp.float32),
            pltpu.VMEM((CH, D_FEAT), jnp.float32),
            pltpu.VMEM_SHARED((N_PAD, D_FEAT), jnp.float32),
            pltpu.SemaphoreType.DMA,
            pltpu.SemaphoreType.DMA,
            pltpu.SemaphoreType.DMA,
            pltpu.SemaphoreType.DMA,
        ],
    )
    def k(ef_hbm, ei_hbm, zeros_hbm, out_hbm, ir0, ir1, b0, b1, acc,
          isem0, isem1, rsem0, rsem1):
        c = lax.axis_index("c")
        s = lax.axis_index("s")
        wid = s * NC + c
        r0 = s * STRIPE
        cbase = wid * CPT
        irs, bufs = (ir0, ir1), (b0, b1)
        isems, rsems = (isem0, isem1), (rsem0, rsem1)

        def idx_slice(cid):
            return ei_hbm.at[0, pl.ds(cid * CH, CH)]

        def edge_slice(cid):
            return ef_hbm.at[pl.ds(cid * CH, CH)]

        for h in range(2):
            pltpu.async_copy(idx_slice(cbase + h), irs[h], isems[h])
            pltpu.async_copy(edge_slice(cbase + h), bufs[h], rsems[h])
        pltpu.sync_copy(zeros_hbm, acc.at[pl.ds(r0, STRIPE)])
        plsc.subcore_barrier()

        def body(p, carry):
            for h in range(2):
                j = 2 * p + h
                cid = cbase + j
                ir, buf = irs[h], bufs[h]
                pltpu.make_async_copy(idx_slice(cid), ir, isems[h]).wait()
                pltpu.make_async_copy(edge_slice(cid), buf, rsems[h]).wait()
                pltpu.sync_copy(buf, acc.at[ir], add=True)
                @pl.when(j + 2 < CPT)
                def _():
                    pltpu.async_copy(idx_slice(cid + 2), ir, isems[h])
                    pltpu.async_copy(edge_slice(cid + 2), buf, rsems[h])
            return carry

        lax.fori_loop(0, CPT // 2, body, 0)

        @pl.when(wid < NEXTRA)
        def _():
            cid = NW * CPT + wid
            pltpu.sync_copy(idx_slice(cid), ir0)
            pltpu.sync_copy(edge_slice(cid), b0)
            pltpu.sync_copy(b0, acc.at[ir0], add=True)

        plsc.subcore_barrier()
        pltpu.sync_copy(acc.at[pl.ds(r0, STRIPE)],
                        out_hbm.at[c, pl.ds(r0, STRIPE)])

    return k(edge_features, edge_index, zeros)


def _tc_combine(partials):
    def body(p_ref, o_ref):
        o_ref[...] = (p_ref[0] + p_ref[1]) * SCALE

    return pl.pallas_call(
        body,
        out_shape=jax.ShapeDtypeStruct((N_NODES, D_FEAT), jnp.float32),
        grid=(2,),
        in_specs=[pl.BlockSpec((2, 5000, D_FEAT), lambda i: (0, i, 0))],
        out_specs=pl.BlockSpec((5000, D_FEAT), lambda i: (i, 0)),
    )(partials)


def kernel(edge_features, edge_index, pos):
    zeros = jnp.zeros((STRIPE, D_FEAT), jnp.float32)
    partials = _sc_scatter_partials(edge_features, edge_index, zeros)
    return _tc_combine(partials)
